# baseline (speedup 1.0000x reference)
import functools

import jax
import jax.numpy as jnp
from jax.experimental import pallas as pl
from jax.experimental.pallas import tpu as pltpu

_B, _H, _S, _D = 8, 16, 2048, 128
_Q = 16
_BH = _B * _H
_R = 4


def _update_body(pos_ref, kc_ref, vc_ref, kv_ref, vv_ref, ko_ref, vo_ref):
    ko_ref[...] = kc_ref[...]
    vo_ref[...] = vc_ref[...]
    row_iota = jax.lax.broadcasted_iota(jnp.int32, (1, 8, 1), 1)
    for q in range(_Q):
        p = pos_ref[q]
        p8 = pl.multiple_of((p // 8) * 8, 8)
        mask = row_iota == (p - p8)
        ko_ref[:, pl.ds(p8, 8), :] = jnp.where(
            mask, kv_ref[:, pl.ds(q, 1), :], ko_ref[:, pl.ds(p8, 8), :]
        )
        vo_ref[:, pl.ds(p8, 8), :] = jnp.where(
            mask, vv_ref[:, pl.ds(q, 1), :], vo_ref[:, pl.ds(p8, 8), :]
        )


@jax.jit
def kernel(k_cache, v_cache, input_pos, k_val, v_val):
    kc = k_cache.reshape(_BH, _S, _D)
    vc = v_cache.reshape(_BH, _S, _D)
    kv = k_val.reshape(_BH, _Q, _D)
    vv = v_val.reshape(_BH, _Q, _D)

    grid = (_BH // _R,)
    cache_spec = pl.BlockSpec((_R, _S, _D), lambda i, pos: (i, 0, 0))
    val_spec = pl.BlockSpec((_R, _Q, _D), lambda i, pos: (i, 0, 0))

    ko, vo = pl.pallas_call(
        _update_body,
        grid_spec=pltpu.PrefetchScalarGridSpec(
            num_scalar_prefetch=1,
            grid=grid,
            in_specs=[cache_spec, cache_spec, val_spec, val_spec],
            out_specs=[cache_spec, cache_spec],
        ),
        out_shape=[
            jax.ShapeDtypeStruct((_BH, _S, _D), k_cache.dtype),
            jax.ShapeDtypeStruct((_BH, _S, _D), v_cache.dtype),
        ],
        compiler_params=pltpu.CompilerParams(
            dimension_semantics=("arbitrary",),
        ),
    )(input_pos, kc, vc, kv, vv)

    return (ko.reshape(_B, _H, _S, _D), vo.reshape(_B, _H, _S, _D))


# speedup vs baseline: 1.0200x; 1.0200x over previous
import functools

import jax
import jax.numpy as jnp
from jax.experimental import pallas as pl
from jax.experimental.pallas import tpu as pltpu

_B, _H, _S, _D = 8, 16, 2048, 128
_Q = 16
_BH = _B * _H
_R = 8


def _update_body(pos_ref, kc_ref, vc_ref, kv_ref, vv_ref, ko_ref, vo_ref):
    ko_ref[...] = kc_ref[...]
    vo_ref[...] = vc_ref[...]
    row_iota = jax.lax.broadcasted_iota(jnp.int32, (1, 8, 1), 1)
    for q in range(_Q):
        p = pos_ref[q]
        p8 = pl.multiple_of((p // 8) * 8, 8)
        mask = row_iota == (p - p8)
        ko_ref[:, pl.ds(p8, 8), :] = jnp.where(
            mask, kv_ref[:, pl.ds(q, 1), :], ko_ref[:, pl.ds(p8, 8), :]
        )
        vo_ref[:, pl.ds(p8, 8), :] = jnp.where(
            mask, vv_ref[:, pl.ds(q, 1), :], vo_ref[:, pl.ds(p8, 8), :]
        )


@jax.jit
def kernel(k_cache, v_cache, input_pos, k_val, v_val):
    kc = k_cache.reshape(_BH, _S, _D)
    vc = v_cache.reshape(_BH, _S, _D)
    kv = k_val.reshape(_BH, _Q, _D)
    vv = v_val.reshape(_BH, _Q, _D)

    grid = (_BH // _R,)
    cache_spec = pl.BlockSpec((_R, _S, _D), lambda i, pos: (i, 0, 0))
    val_spec = pl.BlockSpec((_R, _Q, _D), lambda i, pos: (i, 0, 0))

    ko, vo = pl.pallas_call(
        _update_body,
        grid_spec=pltpu.PrefetchScalarGridSpec(
            num_scalar_prefetch=1,
            grid=grid,
            in_specs=[cache_spec, cache_spec, val_spec, val_spec],
            out_specs=[cache_spec, cache_spec],
        ),
        out_shape=[
            jax.ShapeDtypeStruct((_BH, _S, _D), k_cache.dtype),
            jax.ShapeDtypeStruct((_BH, _S, _D), v_cache.dtype),
        ],
        compiler_params=pltpu.CompilerParams(
            dimension_semantics=("arbitrary",),
        ),
    )(input_pos, kc, vc, kv, vv)

    return (ko.reshape(_B, _H, _S, _D), vo.reshape(_B, _H, _S, _D))
